# Initial kernel scaffold; baseline (speedup 1.0000x reference)
import jax
import jax.numpy as jnp
from jax.experimental import pallas as pl


def kernel(u_entity, u_heads, u_relations, u_tails, i_entity, i_heads, i_relations, i_tails, entity_emb, rel_emb, W1, b1, W2, b2, W3, b3, Wagg, bagg):
    raise NotImplementedError("write your pallas kernel here")


# speedup vs baseline: 5.4438x; 5.4438x over previous
import functools

import jax
import jax.numpy as jnp
from jax import lax
from jax.experimental import pallas as pl
from jax.experimental.pallas import tpu as pltpu
from jax.experimental.pallas import tpu_sc as plsc

N_ENT = 100000
N_REL = 32
DIM = 128
L = 2
N = 1024
K = 64

NB = 32
R = NB * K
GRID = N // NB
NGATH = 10 * N * K


def _make_sc_gather(B, D, C):
    info = plsc.get_sparse_core_info()
    NC, NS = info.num_cores, info.num_subcores
    NW = NC * NS
    per_w = B // NW
    n_chunks = per_w // C
    assert per_w % C == 0 and B % NW == 0

    mesh = plsc.VectorSubcoreMesh(core_axis_name="c", subcore_axis_name="s")

    @functools.partial(
        pl.kernel,
        mesh=mesh,
        out_type=jax.ShapeDtypeStruct((B, D), jnp.float32),
        scratch_types=[
            pltpu.VMEM((C,), jnp.int32),
            pltpu.VMEM((C, D), jnp.float32),
            pltpu.SemaphoreType.DMA,
        ],
    )
    def gather_k(table_hbm, idx_hbm, out_hbm, idx_v, rows_v, sem):
        wid = lax.axis_index("s") * NC + lax.axis_index("c")
        base = wid * per_w

        def body(i, carry):
            off = base + i * C
            pltpu.sync_copy(idx_hbm.at[pl.ds(off, C)], idx_v)
            pltpu.async_copy(table_hbm.at[idx_v], rows_v, sem).wait()
            pltpu.sync_copy(rows_v, out_hbm.at[pl.ds(off, C)])
            return carry

        lax.fori_loop(0, n_chunks, body, 0)

    return gather_k


@functools.lru_cache(maxsize=1)
def _sc_gather_cached():
    return _make_sc_gather(NGATH, DIM, 512)


def _tc_body(ur0, ur1, ir0, ir1,
             gue, gie, guh0, guh1, gih0, gih1, gut0, gut1, git0, git1,
             rel_emb, w1u, w1l, w2, w3t, b1, b2, b3, wagg, bagg, out_ref):
    f32 = jnp.float32
    relW = jnp.dot(rel_emb[...], w1l[...])
    jj = lax.broadcasted_iota(jnp.int32, (NB, R), 1)
    nn = lax.broadcasted_iota(jnp.int32, (NB, R), 0)
    seg = ((jj >> 6) == nn).astype(f32)
    segk = seg * (1.0 / K)
    cc = lax.broadcasted_iota(jnp.int32, (N_REL, R), 0)
    w1u_ = w1u[...]
    w2_ = w2[...]
    w3t_ = w3t[...]
    b1_ = b1[...]
    b2_ = b2[...]
    b3_ = b3[0:1, 0:1]
    bagg_ = bagg[...]
    wagg_ = wagg[...]

    def side(ent, h0, h1, t0, t1, r0, r1):
        e0 = jnp.dot(segk, ent[...])
        acc = jnp.dot(e0, wagg_[0:DIM, :])
        for li, (h_ref, t_ref, r_ref) in enumerate(((h0, t0, r0), (h1, t1, r1))):
            rrow = r_ref[...].reshape(1, R)
            ohT = (cc == rrow).astype(f32)
            rb = lax.dot_general(ohT, relW, (((0,), (0,)), ((), ())))
            y = jnp.maximum(jnp.dot(h_ref[...], w1u_) + rb + b1_, 0.0)
            y = jnp.maximum(jnp.dot(y, w2_) + b2_, 0.0)
            lg = lax.dot_general(w3t_, y, (((1,), (1,)), ((), ())))
            ez = jnp.exp(jax.nn.sigmoid(lg + b3_))
            wp = seg * ez
            num = jnp.dot(wp, t_ref[...])
            den = jnp.sum(wp, axis=1, keepdims=True)
            el = num / den
            acc = acc + jnp.dot(el, wagg_[(li + 1) * DIM:(li + 2) * DIM, :])
        return jax.nn.sigmoid(acc + bagg_)

    ue = side(gue, guh0, guh1, gut0, gut1, ur0, ur1)
    ie = side(gie, gih0, gih1, git0, git1, ir0, ir1)
    prod = ue * ie
    ones = jnp.ones((1, DIM), f32)
    v = lax.dot_general(ones, prod, (((1,), (1,)), ((), ())))
    out_ref[0] = jax.nn.sigmoid(v)


def _rel_spec(l):
    return pl.BlockSpec((1, 1, R), lambda n, l=l: (l * GRID + n, 0, 0))


def _gath_spec(region):
    return pl.BlockSpec((R, DIM), lambda n, r=region: (r * GRID + n, 0))


def _w_spec(shape):
    nd = len(shape)
    return pl.BlockSpec(shape, lambda n, _z=(0,) * nd: _z)


def _tc_forward(u_rel3, i_rel3, gath, rel_emb, w1u, w1l, w2, w3t,
                b1, b2, b3, wagg, bagg):
    in_specs = (
        [_rel_spec(0), _rel_spec(1), _rel_spec(0), _rel_spec(1)]
        + [_gath_spec(r) for r in range(10)]
        + [_w_spec(rel_emb.shape), _w_spec(w1u.shape), _w_spec(w1l.shape),
           _w_spec(w2.shape), _w_spec(w3t.shape), _w_spec(b1.shape),
           _w_spec(b2.shape), _w_spec(b3.shape), _w_spec(wagg.shape),
           _w_spec(bagg.shape)]
    )
    out = pl.pallas_call(
        _tc_body,
        grid=(GRID,),
        in_specs=in_specs,
        out_specs=pl.BlockSpec((1, 1, NB), lambda n: (n, 0, 0)),
        out_shape=jax.ShapeDtypeStruct((GRID, 1, NB), jnp.float32),
    )(u_rel3, u_rel3, i_rel3, i_rel3,
      gath, gath, gath, gath, gath, gath, gath, gath, gath, gath,
      rel_emb, w1u, w1l, w2, w3t, b1, b2, b3, wagg, bagg)
    return out.reshape(N)


def kernel(u_entity, u_heads, u_relations, u_tails, i_entity, i_heads,
           i_relations, i_tails, entity_emb, rel_emb, W1, b1, W2, b2, W3, b3,
           Wagg, bagg):
    idx = jnp.concatenate([
        u_entity.reshape(-1), i_entity.reshape(-1),
        u_heads.reshape(-1), i_heads.reshape(-1),
        u_tails.reshape(-1), i_tails.reshape(-1),
    ]).astype(jnp.int32)
    gath = _sc_gather_cached()(entity_emb, idx)

    u_rel3 = u_relations.reshape(L * GRID, 1, R).astype(jnp.int32)
    i_rel3 = i_relations.reshape(L * GRID, 1, R).astype(jnp.int32)
    w1u = W1[:DIM, :]
    w1l = W1[DIM:, :]
    w3t = W3.reshape(1, DIM)
    b1v = b1.reshape(1, DIM)
    b2v = b2.reshape(1, DIM)
    b3v = jnp.broadcast_to(b3.reshape(1, 1), (1, DIM))
    baggv = bagg.reshape(1, DIM)
    return _tc_forward(u_rel3, i_rel3, gath, rel_emb, w1u, w1l, W2, w3t,
                       b1v, b2v, b3v, Wagg, baggv)
